# baseline (speedup 1.0000x reference)
import functools

import jax
import jax.numpy as jnp
from jax import lax
from jax.experimental import pallas as pl
from jax.experimental.pallas import tpu as pltpu
from jax.experimental.pallas import tpu_sc as plsc

N, T, C, P = 16, 2048, 500, 100
L = 16
PG = 7
PPAD = PG * L
TB = 16
NW = 32
THALF = T // 2
NCHUNK = THALF // TB

_MESH = plsc.VectorSubcoreMesh(
    core_axis_name="c", subcore_axis_name="s", num_cores=2, num_subcores=16
)


def _stage1_body(nnet_hbm, labels_hbm, partials_hbm, rows_v, labs_v, acc_v):
    c = lax.axis_index("c")
    s = lax.axis_index("s")
    n = s
    t0base = c * THALF

    def chunk(i, accs):
        t0 = t0base + i * TB
        pltpu.sync_copy(nnet_hbm.at[n, pl.ds(t0, TB), :], rows_v)
        pltpu.sync_copy(labels_hbm.at[n, pl.ds(t0, TB), :], labs_v)
        new = list(accs)
        for t in range(TB):
            tvec = jnp.full((L,), t, jnp.int32)
            for j in range(PG):
                lab = labs_v[t, pl.ds(j * L, L)]
                g = plsc.load_gather(rows_v, [tvec, lab])
                new[j] = new[j] + g
        return tuple(new)

    zero = jnp.zeros((L,), jnp.float32)
    accs = lax.fori_loop(0, NCHUNK, chunk, (zero,) * PG)
    for j in range(PG):
        acc_v[pl.ds(j * L, L)] = accs[j]
    wid = s * 2 + c
    pltpu.sync_copy(acc_v, partials_hbm.at[wid])


def _stage2_body(partials_hbm, wers_hbm, out_hbm, part_v, wers_v, out_v):
    c = lax.axis_index("c")
    s = lax.axis_index("s")
    wid = s * 2 + c

    @pl.when(wid == 0)
    def _():
        pltpu.sync_copy(partials_hbm, part_v)
        pltpu.sync_copy(wers_hbm, wers_v)
        lane = lax.iota(jnp.int32, L)
        neg = jnp.full((L,), -3.0e38, jnp.float32)
        total = jnp.zeros((L,), jnp.float32)
        for n in range(N):
            svecs = []
            for j in range(PG):
                sv = part_v[2 * n, pl.ds(j * L, L)] + part_v[2 * n + 1, pl.ds(j * L, L)]
                msk = (lane + (j * L)) < P
                svecs.append(jnp.where(msk, sv, neg))
            mvec = svecs[0]
            for j in range(1, PG):
                mvec = jnp.maximum(mvec, svecs[j])
            mmax = jnp.max(mvec)
            den = jnp.zeros((L,), jnp.float32)
            num = jnp.zeros((L,), jnp.float32)
            for j in range(PG):
                e = jnp.exp(svecs[j] - mmax)
                den = den + e
                num = num + e * wers_v[n, pl.ds(j * L, L)]
            numsum = jnp.full((L,), jnp.sum(num), jnp.float32)
            densum = jnp.full((L,), jnp.sum(den), jnp.float32)
            total = total + numsum / densum
        out_v[...] = total * (1.0 / (N * P))
        pltpu.sync_copy(out_v, out_hbm)


_stage1 = functools.partial(
    pl.kernel,
    out_type=jax.ShapeDtypeStruct((NW, PPAD), jnp.float32),
    mesh=_MESH,
    compiler_params=pltpu.CompilerParams(
        use_tc_tiling_on_sc=False, needs_layout_passes=False
    ),
    scratch_types=[
        pltpu.VMEM((TB, C), jnp.float32),
        pltpu.VMEM((TB, PPAD), jnp.int32),
        pltpu.VMEM((PPAD,), jnp.float32),
    ],
)(_stage1_body)

_stage2 = functools.partial(
    pl.kernel,
    out_type=jax.ShapeDtypeStruct((L,), jnp.float32),
    mesh=_MESH,
    compiler_params=pltpu.CompilerParams(
        use_tc_tiling_on_sc=False, needs_layout_passes=False
    ),
    scratch_types=[
        pltpu.VMEM((NW, PPAD), jnp.float32),
        pltpu.VMEM((N, PPAD), jnp.float32),
        pltpu.VMEM((L,), jnp.float32),
    ],
)(_stage2_body)


def kernel(nnet_output, path_labels, wers):
    labels_t = jnp.transpose(path_labels.astype(jnp.int32), (0, 2, 1))
    labels_t = jnp.pad(labels_t, ((0, 0), (0, 0), (0, PPAD - P)))
    wers_f = jnp.pad(wers.astype(jnp.float32), ((0, 0), (0, PPAD - P)))
    partials = _stage1(nnet_output, labels_t)
    out = _stage2(partials, wers_f)
    return out[0]


# speedup vs baseline: 1.5102x; 1.5102x over previous
import functools

import jax
import jax.numpy as jnp
from jax import lax
from jax.experimental import pallas as pl
from jax.experimental.pallas import tpu as pltpu
from jax.experimental.pallas import tpu_sc as plsc

N, T, C, P = 16, 2048, 500, 100
L = 16
PG = 7
PPAD = PG * L
TB = 16
NW = 32
THALF = T // 2
NCHUNK = THALF // TB

_MESH = plsc.VectorSubcoreMesh(
    core_axis_name="c", subcore_axis_name="s", num_cores=2, num_subcores=16
)
_PARAMS = pltpu.CompilerParams(
    use_tc_tiling_on_sc=False, needs_layout_passes=False
)


def _stage1_body(nnet_hbm, labels_hbm, partials_hbm, rows2_v, labs2_v, acc_v,
                 sem_r0, sem_r1, sem_l0, sem_l1):
    c = lax.axis_index("c")
    s = lax.axis_index("s")
    n = s
    t0base = c * THALF
    sem_r = (sem_r0, sem_r1)
    sem_l = (sem_l0, sem_l1)

    zero16i = jnp.zeros((L,), jnp.int32)
    for slot in range(2):
        for r in range(P, PPAD):
            labs2_v[slot, r, :] = zero16i

    def dma_pair(chunk, slot):
        t0 = t0base + chunk * TB
        rcp = pltpu.make_async_copy(
            nnet_hbm.at[pl.ds((n * T + t0) * C, TB * C)],
            rows2_v.at[slot], sem_r[slot])
        lcp = pltpu.make_async_copy(
            labels_hbm.at[n, :, pl.ds(t0, TB)],
            labs2_v.at[slot, pl.ds(0, P), :], sem_l[slot])
        return rcp, lcp

    def start(chunk, slot):
        rcp, lcp = dma_pair(chunk, slot)
        rcp.start()
        lcp.start()

    def wait(chunk, slot):
        rcp, lcp = dma_pair(chunk, slot)
        rcp.wait()
        lcp.wait()

    def compute(slot, accs):
        new = list(accs)
        rslot = rows2_v.at[slot]
        lslot = labs2_v.at[slot]
        for t in range(TB):
            tv = jnp.full((L,), t, jnp.int32)
            tc = jnp.full((L,), t * C, jnp.int32)
            for j in range(PG):
                rows_j = lax.iota(jnp.int32, L) + (j * L)
                lab = plsc.load_gather(lslot, [rows_j, tv])
                g = plsc.load_gather(rslot, [lab + tc])
                new[j] = new[j] + g
        return tuple(new)

    start(0, 0)
    start(1, 1)

    def body(k, accs):
        c0 = 2 * k
        wait(c0, 0)
        accs = compute(0, accs)
        start(c0 + 2, 0)
        wait(c0 + 1, 1)
        accs = compute(1, accs)
        start(c0 + 3, 1)
        return accs

    zero = jnp.zeros((L,), jnp.float32)
    accs = lax.fori_loop(0, NCHUNK // 2 - 1, body, (zero,) * PG)
    wait(NCHUNK - 2, 0)
    accs = compute(0, accs)
    wait(NCHUNK - 1, 1)
    accs = compute(1, accs)

    for j in range(PG):
        acc_v[pl.ds(j * L, L)] = accs[j]
    wid = s * 2 + c
    pltpu.sync_copy(acc_v, partials_hbm.at[pl.ds(wid * PPAD, PPAD)])


def _stage2_body(partials_hbm, wers_hbm, out_hbm, part_v, wers_v, out_v):
    c = lax.axis_index("c")
    s = lax.axis_index("s")
    wid = s * 2 + c

    @pl.when(wid == 0)
    def _():
        pltpu.sync_copy(partials_hbm, part_v)
        pltpu.sync_copy(wers_hbm, wers_v)
        lane = lax.iota(jnp.int32, L)
        neg = jnp.full((L,), -3.0e38, jnp.float32)
        total = jnp.zeros((L,), jnp.float32)
        for n in range(N):
            svecs = []
            for j in range(PG):
                sv = (part_v[pl.ds(2 * n * PPAD + j * L, L)]
                      + part_v[pl.ds((2 * n + 1) * PPAD + j * L, L)])
                msk = (lane + (j * L)) < P
                svecs.append(jnp.where(msk, sv, neg))
            mvec = svecs[0]
            for j in range(1, PG):
                mvec = jnp.maximum(mvec, svecs[j])
            mmax = jnp.max(mvec)
            den = jnp.zeros((L,), jnp.float32)
            num = jnp.zeros((L,), jnp.float32)
            for j in range(PG):
                e = jnp.exp(svecs[j] - mmax)
                den = den + e
                num = num + e * wers_v[pl.ds(n * PPAD + j * L, L)]
            numsum = jnp.full((L,), jnp.sum(num), jnp.float32)
            densum = jnp.full((L,), jnp.sum(den), jnp.float32)
            total = total + numsum / densum
        out_v[...] = total * (1.0 / (N * P))
        pltpu.sync_copy(out_v, out_hbm)


_stage1 = functools.partial(
    pl.kernel,
    out_type=jax.ShapeDtypeStruct((NW * PPAD,), jnp.float32),
    mesh=_MESH,
    compiler_params=_PARAMS,
    scratch_types=[
        pltpu.VMEM((2, TB * C), jnp.float32),
        pltpu.VMEM((2, PPAD, TB), jnp.int32),
        pltpu.VMEM((PPAD,), jnp.float32),
        pltpu.SemaphoreType.DMA,
        pltpu.SemaphoreType.DMA,
        pltpu.SemaphoreType.DMA,
        pltpu.SemaphoreType.DMA,
    ],
)(_stage1_body)

_stage2 = functools.partial(
    pl.kernel,
    out_type=jax.ShapeDtypeStruct((L,), jnp.float32),
    mesh=_MESH,
    compiler_params=_PARAMS,
    scratch_types=[
        pltpu.VMEM((NW * PPAD,), jnp.float32),
        pltpu.VMEM((N * PPAD,), jnp.float32),
        pltpu.VMEM((L,), jnp.float32),
    ],
)(_stage2_body)


def kernel(nnet_output, path_labels, wers):
    nnet1 = nnet_output.reshape(-1)
    labels_i = path_labels.astype(jnp.int32)
    wers_f = jnp.pad(
        wers.astype(jnp.float32), ((0, 0), (0, PPAD - P))).reshape(-1)
    partials = _stage1(nnet1, labels_i)
    out = _stage2(partials, wers_f)
    return out[0]
